# baseline (speedup 1.0000x reference)
import functools

import jax
import jax.numpy as jnp
from jax.experimental import pallas as pl
from jax.experimental.pallas import tpu as pltpu


def _ca_kernel(x_ref, w1_ref, w2_ref, o_ref, *, inv_hw):
    x = x_ref[...]
    pooled = jnp.sum(x, axis=(-2, -1)) * inv_hw
    h = jax.lax.dot_general(
        pooled, w1_ref[...], (((1,), (1,)), ((), ())),
        preferred_element_type=jnp.float32)
    h = jnp.maximum(h, 0.0)
    s = jax.lax.dot_general(
        h, w2_ref[...], (((1,), (1,)), ((), ())),
        preferred_element_type=jnp.float32)
    s = jax.nn.sigmoid(s)
    o_ref[...] = x * s[:, :, None, None]


def kernel(x, w1, w2):
    N, C, H, W = x.shape
    Cr = w1.shape[0]

    bn = 1
    assert N % bn == 0

    return pl.pallas_call(
        functools.partial(_ca_kernel, inv_hw=1.0 / (H * W)),
        out_shape=jax.ShapeDtypeStruct((N, C, H, W), x.dtype),
        grid=(N // bn,),
        in_specs=[
            pl.BlockSpec((bn, C, H, W), lambda n: (n, 0, 0, 0)),
            pl.BlockSpec((Cr, C), lambda n: (0, 0)),
            pl.BlockSpec((C, Cr), lambda n: (0, 0)),
        ],
        out_specs=pl.BlockSpec((bn, C, H, W), lambda n: (n, 0, 0, 0)),
        compiler_params=pltpu.CompilerParams(
            dimension_semantics=("parallel",)),
    )(x, w1, w2)


# speedup vs baseline: 6.9206x; 6.9206x over previous
import functools

import jax
import jax.numpy as jnp
from jax.experimental import pallas as pl
from jax.experimental.pallas import tpu as pltpu


def _ca_kernel(x_ref, w1_ref, w2_ref, o_ref, *, inv_hw):
    x = x_ref[...]
    pooled = jnp.sum(x, axis=(1, 2)) * inv_hw
    h = jax.lax.dot_general(
        pooled, w1_ref[...], (((1,), (1,)), ((), ())),
        preferred_element_type=jnp.float32)
    h = jnp.maximum(h, 0.0)
    s = jax.lax.dot_general(
        h, w2_ref[...], (((1,), (1,)), ((), ())),
        preferred_element_type=jnp.float32)
    s = jax.nn.sigmoid(s)
    o_ref[...] = x * s[:, None, None, :]


def kernel(x, w1, w2):
    N, C, H, W = x.shape
    Cr = w1.shape[0]

    xt = jnp.transpose(x, (0, 2, 3, 1))

    bn = 2
    assert N % bn == 0

    out_t = pl.pallas_call(
        functools.partial(_ca_kernel, inv_hw=1.0 / (H * W)),
        out_shape=jax.ShapeDtypeStruct((N, H, W, C), x.dtype),
        grid=(N // bn,),
        in_specs=[
            pl.BlockSpec((bn, H, W, C), lambda n: (n, 0, 0, 0)),
            pl.BlockSpec((Cr, C), lambda n: (0, 0)),
            pl.BlockSpec((C, Cr), lambda n: (0, 0)),
        ],
        out_specs=pl.BlockSpec((bn, H, W, C), lambda n: (n, 0, 0, 0)),
        compiler_params=pltpu.CompilerParams(
            dimension_semantics=("parallel",)),
    )(xt, w1, w2)

    return jnp.transpose(out_t, (0, 3, 1, 2))


# speedup vs baseline: 7.1514x; 1.0334x over previous
import functools

import jax
import jax.numpy as jnp
from jax.experimental import pallas as pl
from jax.experimental.pallas import tpu as pltpu


def _ca_kernel(x_ref, w1_ref, w2_ref, o_ref, *, inv_hw):
    x = x_ref[...]
    pooled = jnp.sum(x, axis=(1, 2)) * inv_hw
    h = jax.lax.dot_general(
        pooled, w1_ref[...], (((1,), (1,)), ((), ())),
        preferred_element_type=jnp.float32)
    h = jnp.maximum(h, 0.0)
    s = jax.lax.dot_general(
        h, w2_ref[...], (((1,), (1,)), ((), ())),
        preferred_element_type=jnp.float32)
    s = jax.nn.sigmoid(s)
    o_ref[...] = x * s[:, None, None, :]


def kernel(x, w1, w2):
    N, C, H, W = x.shape
    Cr = w1.shape[0]

    xt = jnp.transpose(x, (0, 2, 3, 1))

    bn = 4
    assert N % bn == 0

    out_t = pl.pallas_call(
        functools.partial(_ca_kernel, inv_hw=1.0 / (H * W)),
        out_shape=jax.ShapeDtypeStruct((N, H, W, C), x.dtype),
        grid=(N // bn,),
        in_specs=[
            pl.BlockSpec((bn, H, W, C), lambda n: (n, 0, 0, 0)),
            pl.BlockSpec((Cr, C), lambda n: (0, 0)),
            pl.BlockSpec((C, Cr), lambda n: (0, 0)),
        ],
        out_specs=pl.BlockSpec((bn, H, W, C), lambda n: (n, 0, 0, 0)),
        compiler_params=pltpu.CompilerParams(
            dimension_semantics=("parallel",)),
    )(xt, w1, w2)

    return jnp.transpose(out_t, (0, 3, 1, 2))


# speedup vs baseline: 7.1610x; 1.0013x over previous
import functools

import jax
import jax.numpy as jnp
from jax.experimental import pallas as pl
from jax.experimental.pallas import tpu as pltpu


def _ca_kernel(x_ref, w1_ref, w2_ref, o_ref, *, inv_hw):
    x = x_ref[...]
    pooled = jnp.sum(x, axis=(1, 2)) * inv_hw
    h = jax.lax.dot_general(
        pooled, w1_ref[...], (((1,), (1,)), ((), ())),
        preferred_element_type=jnp.float32)
    h = jnp.maximum(h, 0.0)
    s = jax.lax.dot_general(
        h, w2_ref[...], (((1,), (1,)), ((), ())),
        preferred_element_type=jnp.float32)
    s = jax.nn.sigmoid(s)
    o_ref[...] = x * s[:, None, None, :]


def kernel(x, w1, w2):
    N, C, H, W = x.shape
    Cr = w1.shape[0]

    xt = jnp.transpose(x, (0, 2, 3, 1))

    bn = next(b for b in (4, 2, 1) if N % b == 0)

    out_t = pl.pallas_call(
        functools.partial(_ca_kernel, inv_hw=1.0 / (H * W)),
        out_shape=jax.ShapeDtypeStruct((N, H, W, C), x.dtype),
        grid=(N // bn,),
        in_specs=[
            pl.BlockSpec((bn, H, W, C), lambda n: (n, 0, 0, 0)),
            pl.BlockSpec((Cr, C), lambda n: (0, 0)),
            pl.BlockSpec((C, Cr), lambda n: (0, 0)),
        ],
        out_specs=pl.BlockSpec((bn, H, W, C), lambda n: (n, 0, 0, 0)),
        compiler_params=pltpu.CompilerParams(
            dimension_semantics=("parallel",)),
    )(xt, w1, w2)

    return jnp.transpose(out_t, (0, 3, 1, 2))
